# baseline (speedup 1.0000x reference)
import jax
import jax.numpy as jnp
from jax.experimental import pallas as pl
from jax.experimental.pallas import tpu as pltpu

N = 1024
D = 128
CHUNK = 256
NCHUNK = N // CHUNK


def _gin_kernel(a_any, x_any, x_ref, w_ref, b_ref, o_any):
    def body(a_chunk, x_chunk, o_chunk):
        aggr = jnp.dot(a_chunk[...], x_ref[...],
                       preferred_element_type=jnp.float32)
        h = aggr + x_chunk[...]
        out = jax.lax.dot_general(h, w_ref[...], (((1,), (1,)), ((), ())),
                                  preferred_element_type=jnp.float32)
        o_chunk[...] = jnp.maximum(out + b_ref[...], 0.0)

    pltpu.emit_pipeline(
        body,
        grid=(NCHUNK,),
        in_specs=[
            pl.BlockSpec((CHUNK, N), lambda i: (i, 0)),
            pl.BlockSpec((CHUNK, D), lambda i: (i, 0)),
        ],
        out_specs=[pl.BlockSpec((CHUNK, D), lambda i: (i, 0))],
    )(a_any, x_any, o_any)


def kernel(A, X, W, b):
    return pl.pallas_call(
        _gin_kernel,
        in_specs=[
            pl.BlockSpec(memory_space=pl.ANY),
            pl.BlockSpec(memory_space=pl.ANY),
            pl.BlockSpec((N, D), lambda: (0, 0)),
            pl.BlockSpec((D, D), lambda: (0, 0)),
            pl.BlockSpec((1, D), lambda: (0, 0)),
        ],
        out_specs=pl.BlockSpec(memory_space=pl.ANY),
        out_shape=jax.ShapeDtypeStruct((N, D), jnp.float32),
    )(A, X, X, W, b.reshape(1, D))


# speedup vs baseline: 1.6185x; 1.6185x over previous
import jax
import jax.numpy as jnp
from jax.experimental import pallas as pl
from jax.experimental.pallas import tpu as pltpu

N = 1024
D = 128
BM = 512


def _gin_kernel(a_ref, x_ref, w_ref, b_ref, o_ref, z_ref):
    i = pl.program_id(0)

    @pl.when(i == 0)
    def _():
        z_ref[...] = jax.lax.dot_general(
            x_ref[...], w_ref[...], (((1,), (1,)), ((), ())),
            preferred_element_type=jnp.float32)

    aggr = jnp.dot(a_ref[...], z_ref[...], preferred_element_type=jnp.float32)
    o_ref[...] = jnp.maximum(
        aggr + z_ref[pl.ds(i * BM, BM), :] + b_ref[...], 0.0)


def kernel(A, X, W, b):
    return pl.pallas_call(
        _gin_kernel,
        grid=(N // BM,),
        in_specs=[
            pl.BlockSpec((BM, N), lambda i: (i, 0)),
            pl.BlockSpec((N, D), lambda i: (0, 0)),
            pl.BlockSpec((D, D), lambda i: (0, 0)),
            pl.BlockSpec((1, D), lambda i: (0, 0)),
        ],
        out_specs=pl.BlockSpec((BM, D), lambda i: (i, 0)),
        out_shape=jax.ShapeDtypeStruct((N, D), jnp.float32),
        scratch_shapes=[pltpu.VMEM((N, D), jnp.float32)],
    )(A, X, W, b.reshape(1, D))
